# Initial kernel scaffold; baseline (speedup 1.0000x reference)
import jax
import jax.numpy as jnp
from jax.experimental import pallas as pl


def kernel(emb_weight, l_q, l_k):
    raise NotImplementedError("write your pallas kernel here")


# speedup vs baseline: 42.5925x; 42.5925x over previous
import functools

import jax
import jax.numpy as jnp
from jax import lax
from jax.experimental import pallas as pl
from jax.experimental.pallas import tpu as pltpu
from jax.experimental.pallas import tpu_sc as plsc

H = 16
L_Q = 2048
L_K = 2048
EXT = 4096


@functools.partial(
    pl.kernel,
    out_type=jax.ShapeDtypeStruct((H * L_Q, L_K), jnp.float32),
    mesh=plsc.VectorSubcoreMesh(core_axis_name="c", subcore_axis_name="s"),
    compiler_params=pltpu.CompilerParams(
        needs_layout_passes=False, use_tc_tiling_on_sc=False
    ),
    scratch_types=[
        pltpu.VMEM((520,), jnp.float32),
        pltpu.VMEM((16,), jnp.int32),
        pltpu.VMEM((8 * EXT,), jnp.float32),
        pltpu.SemaphoreType.DMA,
    ],
)
def _rpe_sc(embT_hbm, dvec_hbm, out_hbm, embrow_v, dvec_v, ext_v, sem):
    c = lax.axis_index("c")
    s = lax.axis_index("s")
    wid = s * 2 + c
    h = wid // 2
    i0 = (wid % 2) * (L_Q // 2)

    pltpu.sync_copy(embT_hbm.at[h], embrow_v)
    pltpu.sync_copy(dvec_hbm, dvec_v)
    vd = dvec_v[...]
    iota = lax.iota(jnp.int32, 16)

    def build(k, carry):
        p = k // (EXT // 16)
        m0 = (k % (EXT // 16)) * 16
        idx = jnp.clip((2303 - m0) - p - iota + vd, 0, 512)
        off = pl.multiple_of(p * EXT + m0, 16)
        ext_v[pl.ds(off, 16)] = plsc.load_gather(embrow_v, [idx])
        return carry

    lax.fori_loop(0, 8 * (EXT // 16), build, 0)

    def blk(b, carry):
        base = i0 + b * 16
        descs = []
        for t in range(16):
            i = base + t
            p = (7 - t) % 8
            aligned = (2047 - base - t) - p
            src = ext_v.at[pl.ds(pl.multiple_of(p * EXT + aligned, 8), L_K)]
            descs.append(pltpu.async_copy(src, out_hbm.at[h * L_Q + i], sem))
        for dsc in descs:
            dsc.wait()
        return carry

    lax.fori_loop(0, (L_Q // 2) // 16, blk, 0)


def kernel(emb_weight, l_q, l_k):
    embT = jnp.transpose(emb_weight).astype(jnp.float32)
    embT = jnp.pad(embT, ((0, 0), (0, 7)))
    d = jnp.asarray(l_q, jnp.int32) - jnp.asarray(l_k, jnp.int32)
    dvec = jnp.broadcast_to(d, (16,)).astype(jnp.int32)
    out = _rpe_sc(embT, dvec)
    return out.reshape(1, H, L_Q, L_K)
